# baseline (speedup 1.0000x reference)
import functools

import jax
import jax.numpy as jnp
from jax import lax
from jax.experimental import pallas as pl
from jax.experimental.pallas import tpu as pltpu
from jax.experimental.pallas import tpu_sc as plsc

NC = 2
NS = 16
NW = NC * NS
K = 80
G = 16
NB = 4


def _pack_rows(a, npad):
    ab = a.astype(jnp.bfloat16).reshape(npad, 4, 2, 16)
    ab = jnp.swapaxes(ab, 2, 3)
    return lax.bitcast_convert_type(ab, jnp.int32).reshape(npad, 64)


def _sc_segment_sum(x, src3, dst3):
    npad = x.shape[0]
    d = 128
    ch = src3.shape[1]
    rps = npad // NS
    assert rps % K == 0 and ch % G == 0

    def body(x_hbm, src_hbm, dst_hbm, sum_out, srcg, dstg, g0, g1, g2,
             rowsf, acc, sm0, sm1, sm2):
        c = lax.axis_index("c")
        s = lax.axis_index("s")
        wid = s * NC + c
        gi = [g0, g1, g2]
        sems = [sm0, sm1, sm2]

        def fz(i, carry):
            for j in range(d // 16):
                rowsf[i, pl.ds(j * 16, 16)] = jnp.zeros((16,), jnp.float32)
            return carry
        lax.fori_loop(0, K, fz, 0)
        for b in range(rps // K):
            pltpu.sync_copy(rowsf, acc.at[pl.ds(s * rps + b * K, K)])
        plsc.subcore_barrier()

        def group(g, carry):
            pltpu.sync_copy(src_hbm.at[wid, pl.ds(g * G, G)], srcg)
            pltpu.sync_copy(dst_hbm.at[wid, pl.ds(g * G, G)], dstg)
            gcps = [pltpu.async_copy(x_hbm.at[srcg.at[j]], gi[j], sems[j])
                    for j in range(3)]
            for j in range(G):
                b = j % 3
                gcps[b].wait()
                gb = gi[b]

                def cvt(i0, carry2):
                    for r in range(4):
                        i = i0 * 4 + r
                        for t in range(4):
                            w = gb[i, pl.ds(t * 16, 16)]
                            lo = lax.bitcast_convert_type(w << 16,
                                                          jnp.float32)
                            hi = lax.bitcast_convert_type(
                                w & jnp.int32(-65536), jnp.float32)
                            rowsf[i, pl.ds(t * 32, 16)] = lo
                            rowsf[i, pl.ds(t * 32 + 16, 16)] = hi
                    return carry2
                lax.fori_loop(0, K // 4, cvt, 0)
                if j + 3 < G:
                    gcps[b] = pltpu.async_copy(
                        x_hbm.at[srcg.at[j + 3]], gi[b], sems[b])
                pltpu.sync_copy(rowsf, acc.at[dstg.at[j]], add=True)
            return carry
        lax.fori_loop(0, ch // G, group, 0)
        plsc.subcore_barrier()

        pltpu.sync_copy(acc.at[pl.ds(s * rps, rps)],
                        sum_out.at[c, pl.ds(s * rps, rps)])

    mesh = plsc.VectorSubcoreMesh(core_axis_name="c", subcore_axis_name="s")
    fn = pl.kernel(
        body,
        out_type=jax.ShapeDtypeStruct((NC, npad, d), jnp.float32),
        mesh=mesh,
        compiler_params=pltpu.CompilerParams(use_tc_tiling_on_sc=False),
        scratch_types=(
            pltpu.VMEM((G, K), jnp.int32),
            pltpu.VMEM((G, K), jnp.int32),
            pltpu.VMEM((K, 64), jnp.int32),
            pltpu.VMEM((K, 64), jnp.int32),
            pltpu.VMEM((K, 64), jnp.int32),
            pltpu.VMEM((K, d), jnp.float32),
            pltpu.VMEM_SHARED((npad, d), jnp.float32),
            pltpu.SemaphoreType.DMA,
            pltpu.SemaphoreType.DMA,
            pltpu.SemaphoreType.DMA,
        ),
    )
    return fn(x, src3, dst3)


def _sc_segment_cnt(dst3, npad):
    ch = dst3.shape[1]
    d = 128
    rps = npad // NS
    assert rps % K == 0

    def body(dst_hbm, cnt_out, dsta, ones_v, cacc, sem):
        c = lax.axis_index("c")
        s = lax.axis_index("s")
        wid = s * NC + c

        def fz(i, carry):
            for j in range(d // 16):
                ones_v[i, pl.ds(j * 16, 16)] = jnp.zeros((16,), jnp.float32)
            return carry
        lax.fori_loop(0, K, fz, 0)
        for b in range(rps // K):
            pltpu.sync_copy(ones_v, cacc.at[pl.ds(s * rps + b * K, K)])

        def fo(i, carry):
            for j in range(d // 16):
                ones_v[i, pl.ds(j * 16, 16)] = jnp.ones((16,), jnp.float32)
            return carry
        lax.fori_loop(0, K, fo, 0)
        plsc.subcore_barrier()

        pltpu.sync_copy(dst_hbm.at[wid], dsta)

        def grp(g, carry):
            cps = [pltpu.async_copy(ones_v, cacc.at[dsta.at[g * 8 + j]], sem,
                                    add=True) for j in range(8)]
            for cp in cps:
                cp.wait()
            return carry
        lax.fori_loop(0, ch // 8, grp, 0)
        plsc.subcore_barrier()

        pltpu.sync_copy(cacc.at[pl.ds(s * rps, rps)],
                        cnt_out.at[c, pl.ds(s * rps, rps)])

    mesh = plsc.VectorSubcoreMesh(core_axis_name="c", subcore_axis_name="s")
    fn = pl.kernel(
        body,
        out_type=jax.ShapeDtypeStruct((NC, npad, d), jnp.float32),
        mesh=mesh,
        scratch_types=(
            pltpu.VMEM((ch, K), jnp.int32),
            pltpu.VMEM((K, d), jnp.float32),
            pltpu.VMEM_SHARED((npad, d), jnp.float32),
            pltpu.SemaphoreType.DMA,
        ),
    )
    return fn(dst3)


def _tc_dense_body(sp_ref, cp_ref, x_ref, wl_ref, wr_ref, b_ref, o_ref, *,
                   relu):
    ssum = sp_ref[0] + sp_ref[1]
    cnt = cp_ref[0, :, 0:1] + cp_ref[1, :, 0:1]
    agg = ssum / jnp.maximum(cnt, 1.0)
    wl = jnp.sign(wl_ref[...])
    wr = jnp.sign(wr_ref[...])
    dn = (((1,), (1,)), ((), ()))
    out = (lax.dot_general(agg, wl, dn, preferred_element_type=jnp.float32,
                           precision=lax.Precision.HIGHEST)
           + lax.dot_general(x_ref[...], wr, dn,
                             preferred_element_type=jnp.float32,
                             precision=lax.Precision.HIGHEST)
           + b_ref[...])
    o_ref[...] = jnp.maximum(out, 0.0) if relu else out


def _tc_dense(sums, cnts, x, w_l, w_r, b_l, relu):
    n, d = x.shape
    o = w_l.shape[0]
    bn = 2048
    grid = (n // bn,)
    return pl.pallas_call(
        functools.partial(_tc_dense_body, relu=relu),
        grid=grid,
        in_specs=[
            pl.BlockSpec((NC, bn, d), lambda i: (0, i, 0)),
            pl.BlockSpec((NC, bn, d), lambda i: (0, i, 0)),
            pl.BlockSpec((bn, d), lambda i: (i, 0)),
            pl.BlockSpec((o, d), lambda i: (0, 0)),
            pl.BlockSpec((o, d), lambda i: (0, 0)),
            pl.BlockSpec((1, o), lambda i: (0, 0)),
        ],
        out_specs=pl.BlockSpec((bn, o), lambda i: (i, 0)),
        out_shape=jax.ShapeDtypeStruct((n, o), jnp.float32),
    )(sums, cnts, x, w_l, w_r, b_l)


def kernel(x, edge_index, W1_l, b1_l, W1_r, W2_l, b2_l, W2_r):
    n = x.shape[0]
    e = edge_index.shape[1]
    npad = ((n + 2047) // 2048) * 2048
    epw = e // NW
    epw_pad = ((epw + K * G - 1) // (K * G)) * (K * G)
    ch = epw_pad // K

    src = edge_index[0].reshape(NW, epw)
    dst = edge_index[1].reshape(NW, epw)
    pad = ((0, 0), (0, epw_pad - epw))
    src3 = jnp.pad(src, pad).reshape(NW, ch, K)
    dst3 = jnp.pad(dst, pad, constant_values=npad - 1).reshape(NW, ch, K)
    xp = jnp.pad(x, ((0, npad - n), (0, 0)))
    b1 = b1_l.reshape(1, -1)
    b2 = b2_l.reshape(1, -1)

    cnt = _sc_segment_cnt(dst3, npad)
    cnt, xp, src3, dst3 = lax.optimization_barrier((cnt, xp, src3, dst3))
    sum1 = _sc_segment_sum(_pack_rows(xp, npad), src3, dst3)
    h = _tc_dense(sum1, cnt, xp, W1_l, W1_r, b1, relu=True)
    sum2 = _sc_segment_sum(_pack_rows(h, npad), src3, dst3)
    out = _tc_dense(sum2, cnt, h, W2_l, W2_r, b2, relu=False)
    return out[:n]


# speedup vs baseline: 1.0684x; 1.0684x over previous
import functools

import jax
import jax.numpy as jnp
from jax import lax
from jax.experimental import pallas as pl
from jax.experimental.pallas import tpu as pltpu
from jax.experimental.pallas import tpu_sc as plsc

NC = 2
NS = 16
NW = NC * NS
K = 80
G = 16
NB = 4


def _sc_segment_sum(x, src3, dst3):
    npad, d = x.shape
    ch = src3.shape[1]
    rps = npad // NS
    assert rps % K == 0 and ch % G == 0

    def body(x_hbm, src_hbm, dst_hbm, sum_out, srcg, dstg, r0, r1, r2, r3,
             acc, sm0, sm1, sm2, sm3, ss0, ss1, ss2, ss3):
        c = lax.axis_index("c")
        s = lax.axis_index("s")
        wid = s * NC + c
        rows = [r0, r1, r2, r3]
        sems = [sm0, sm1, sm2, sm3]
        ssems = [ss0, ss1, ss2, ss3]

        def fz(i, carry):
            for j in range(d // 16):
                r0[i, pl.ds(j * 16, 16)] = jnp.zeros((16,), jnp.float32)
            return carry
        lax.fori_loop(0, K, fz, 0)
        for b in range(rps // K):
            pltpu.sync_copy(r0, acc.at[pl.ds(s * rps + b * K, K)])
        plsc.subcore_barrier()

        def group(g, carry):
            pltpu.sync_copy(src_hbm.at[wid, pl.ds(g * G, G)], srcg)
            pltpu.sync_copy(dst_hbm.at[wid, pl.ds(g * G, G)], dstg)
            gcps = [pltpu.async_copy(x_hbm.at[srcg.at[j]], rows[j], sems[j])
                    for j in range(NB)]
            for j in range(G):
                b = j % NB
                gcps[b].wait()
                pltpu.sync_copy(rows[b], acc.at[dstg.at[j]], add=True)
                if j + NB < G:
                    gcps[b] = pltpu.async_copy(
                        x_hbm.at[srcg.at[j + NB]], rows[b], sems[b])
            return carry
        lax.fori_loop(0, ch // G, group, 0)
        plsc.subcore_barrier()

        pltpu.sync_copy(acc.at[pl.ds(s * rps, rps)],
                        sum_out.at[c, pl.ds(s * rps, rps)])

    mesh = plsc.VectorSubcoreMesh(core_axis_name="c", subcore_axis_name="s")
    fn = pl.kernel(
        body,
        out_type=jax.ShapeDtypeStruct((NC, npad, d), jnp.float32),
        mesh=mesh,
        scratch_types=(
            pltpu.VMEM((G, K), jnp.int32),
            pltpu.VMEM((G, K), jnp.int32),
            pltpu.VMEM((K, d), jnp.float32),
            pltpu.VMEM((K, d), jnp.float32),
            pltpu.VMEM((K, d), jnp.float32),
            pltpu.VMEM((K, d), jnp.float32),
            pltpu.VMEM_SHARED((npad, d), jnp.float32),
            pltpu.SemaphoreType.DMA,
            pltpu.SemaphoreType.DMA,
            pltpu.SemaphoreType.DMA,
            pltpu.SemaphoreType.DMA,
            pltpu.SemaphoreType.DMA,
            pltpu.SemaphoreType.DMA,
            pltpu.SemaphoreType.DMA,
            pltpu.SemaphoreType.DMA,
        ),
    )
    return fn(x, src3, dst3)


def _sc_segment_cnt(dst3, npad):
    ch = dst3.shape[1]
    d = 128
    rps = npad // NS
    assert rps % K == 0

    def body(dst_hbm, cnt_out, dsta, ones_v, cacc, sem):
        c = lax.axis_index("c")
        s = lax.axis_index("s")
        wid = s * NC + c

        def fz(i, carry):
            for j in range(d // 16):
                ones_v[i, pl.ds(j * 16, 16)] = jnp.zeros((16,), jnp.float32)
            return carry
        lax.fori_loop(0, K, fz, 0)
        for b in range(rps // K):
            pltpu.sync_copy(ones_v, cacc.at[pl.ds(s * rps + b * K, K)])

        def fo(i, carry):
            for j in range(d // 16):
                ones_v[i, pl.ds(j * 16, 16)] = jnp.ones((16,), jnp.float32)
            return carry
        lax.fori_loop(0, K, fo, 0)
        plsc.subcore_barrier()

        pltpu.sync_copy(dst_hbm.at[wid], dsta)

        def grp(g, carry):
            cps = [pltpu.async_copy(ones_v, cacc.at[dsta.at[g * 8 + j]], sem,
                                    add=True) for j in range(8)]
            for cp in cps:
                cp.wait()
            return carry
        lax.fori_loop(0, ch // 8, grp, 0)
        plsc.subcore_barrier()

        pltpu.sync_copy(cacc.at[pl.ds(s * rps, rps)],
                        cnt_out.at[c, pl.ds(s * rps, rps)])

    mesh = plsc.VectorSubcoreMesh(core_axis_name="c", subcore_axis_name="s")
    fn = pl.kernel(
        body,
        out_type=jax.ShapeDtypeStruct((NC, npad, d), jnp.float32),
        mesh=mesh,
        scratch_types=(
            pltpu.VMEM((ch, K), jnp.int32),
            pltpu.VMEM((K, d), jnp.float32),
            pltpu.VMEM_SHARED((npad, d), jnp.float32),
            pltpu.SemaphoreType.DMA,
        ),
    )
    return fn(dst3)


def _tc_dense_body(sp_ref, cp_ref, x_ref, wl_ref, wr_ref, b_ref, o_ref, *,
                   relu):
    ssum = sp_ref[0] + sp_ref[1]
    cnt = cp_ref[0, :, 0:1] + cp_ref[1, :, 0:1]
    agg = ssum / jnp.maximum(cnt, 1.0)
    wl = jnp.sign(wl_ref[...])
    wr = jnp.sign(wr_ref[...])
    dn = (((1,), (1,)), ((), ()))
    out = (lax.dot_general(agg, wl, dn, preferred_element_type=jnp.float32,
                           precision=lax.Precision.HIGHEST)
           + lax.dot_general(x_ref[...], wr, dn,
                             preferred_element_type=jnp.float32,
                             precision=lax.Precision.HIGHEST)
           + b_ref[...])
    o_ref[...] = jnp.maximum(out, 0.0) if relu else out


def _tc_dense(sums, cnts, x, w_l, w_r, b_l, relu):
    n, d = x.shape
    o = w_l.shape[0]
    bn = 2048
    grid = (n // bn,)
    return pl.pallas_call(
        functools.partial(_tc_dense_body, relu=relu),
        grid=grid,
        in_specs=[
            pl.BlockSpec((NC, bn, d), lambda i: (0, i, 0)),
            pl.BlockSpec((NC, bn, d), lambda i: (0, i, 0)),
            pl.BlockSpec((bn, d), lambda i: (i, 0)),
            pl.BlockSpec((o, d), lambda i: (0, 0)),
            pl.BlockSpec((o, d), lambda i: (0, 0)),
            pl.BlockSpec((1, o), lambda i: (0, 0)),
        ],
        out_specs=pl.BlockSpec((bn, o), lambda i: (i, 0)),
        out_shape=jax.ShapeDtypeStruct((n, o), jnp.float32),
    )(sums, cnts, x, w_l, w_r, b_l)


def kernel(x, edge_index, W1_l, b1_l, W1_r, W2_l, b2_l, W2_r):
    n = x.shape[0]
    e = edge_index.shape[1]
    npad = ((n + 2047) // 2048) * 2048
    epw = e // NW
    epw_pad = ((epw + K * G - 1) // (K * G)) * (K * G)
    ch = epw_pad // K

    src = edge_index[0].reshape(NW, epw)
    dst = edge_index[1].reshape(NW, epw)
    pad = ((0, 0), (0, epw_pad - epw))
    src3 = jnp.pad(src, pad).reshape(NW, ch, K)
    dst3 = jnp.pad(dst, pad, constant_values=npad - 1).reshape(NW, ch, K)
    xp = jnp.pad(x, ((0, npad - n), (0, 0)))
    b1 = b1_l.reshape(1, -1)
    b2 = b2_l.reshape(1, -1)

    cnt = _sc_segment_cnt(dst3, npad)
    cnt, xp, src3, dst3 = lax.optimization_barrier((cnt, xp, src3, dst3))
    sum1 = _sc_segment_sum(xp, src3, dst3)
    h = _tc_dense(sum1, cnt, xp, W1_l, W1_r, b1, relu=True)
    sum2 = _sc_segment_sum(h, src3, dst3)
    out = _tc_dense(sum2, cnt, h, W2_l, W2_r, b2, relu=False)
    return out[:n]
